# Initial kernel scaffold; baseline (speedup 1.0000x reference)
import jax
import jax.numpy as jnp
from jax.experimental import pallas as pl


def kernel(x, edge_index, Wl1, bl1, Wr1, Wl2, bl2, Wr2, Wl3, bl3, Wr3, g1, b1, g2, b2):
    raise NotImplementedError("write your pallas kernel here")


# speedup vs baseline: 5.3004x; 5.3004x over previous
import functools

import jax
import jax.numpy as jnp
from jax import lax
from jax.experimental import pallas as pl
from jax.experimental.pallas import tpu as pltpu
from jax.experimental.pallas import tpu_sc as plsc

N = 10000
E = 320000
D = 128
NC, NS = 2, 16
NW = NC * NS
NROW = 10240
RPT = NROW // NS
EPW = E // NW
CH = 80
NCHUNK = EPW // CH
RCH = 128
BLK = 1024


def _zero_2d(buf, rows):
    z16 = jnp.zeros((16,), jnp.float32)

    def body(r, c):
        for j in range(D // 16):
            buf[r, pl.ds(j * 16, 16)] = z16
        return c

    lax.fori_loop(0, rows, body, 0)


def _fill_1d(buf, n, val):
    v16 = jnp.full((16,), val, jnp.float32)

    def body(i, c):
        buf[pl.ds(i * 16, 16)] = v16
        return c

    lax.fori_loop(0, n // 16, body, 0)


def _make_agg(with_cnt):
    mesh = plsc.VectorSubcoreMesh(
        core_axis_name="c", subcore_axis_name="s", num_cores=NC, num_subcores=NS
    )
    out_type = [jax.ShapeDtypeStruct((NC * NROW, D), jnp.float32)]
    scratch = [
        pltpu.VMEM_SHARED((NROW, D), jnp.float32),
        pltpu.VMEM((CH,), jnp.int32),
        pltpu.VMEM((CH,), jnp.int32),
        pltpu.VMEM((CH, D), jnp.float32),
        pltpu.VMEM((RCH, D), jnp.float32),
        pltpu.SemaphoreType.DMA,
    ]
    if with_cnt:
        out_type.append(jax.ShapeDtypeStruct((NC * NROW,), jnp.float32))
        scratch += [
            pltpu.VMEM_SHARED((NROW,), jnp.float32),
            pltpu.VMEM((CH,), jnp.float32),
            pltpu.VMEM((RPT,), jnp.float32),
        ]

    def body(h, src, dst, *refs):
        if with_cnt:
            out_agg, out_cnt, acc, sidx, didx, rows, zbuf, sem, cnt, ones, cbuf = refs
        else:
            out_agg, acc, sidx, didx, rows, zbuf, sem = refs
        cid = lax.axis_index("c")
        sid = lax.axis_index("s")
        r0 = sid * RPT

        _zero_2d(zbuf, RCH)
        for j in range(RPT // RCH):
            pltpu.sync_copy(zbuf, acc.at[pl.ds(r0 + j * RCH, RCH)])
        if with_cnt:
            _fill_1d(cbuf, RPT, 0.0)
            pltpu.sync_copy(cbuf, cnt.at[pl.ds(r0, RPT)])
            _fill_1d(ones, CH, 1.0)
        plsc.subcore_barrier()

        base = (cid * NS + sid) * EPW

        def step(i, c):
            off = base + i * CH
            pltpu.sync_copy(src.at[pl.ds(off, CH)], sidx)
            pltpu.sync_copy(dst.at[pl.ds(off, CH)], didx)
            pltpu.async_copy(h.at[sidx], rows, sem).wait()
            pltpu.sync_copy(rows, acc.at[didx], add=True)
            if with_cnt:
                pltpu.sync_copy(ones, cnt.at[didx], add=True)
            return c

        lax.fori_loop(0, NCHUNK, step, 0)
        plsc.subcore_barrier()

        for j in range(RPT // RCH):
            pltpu.sync_copy(acc.at[pl.ds(r0 + j * RCH, RCH)], zbuf)
            pltpu.sync_copy(zbuf, out_agg.at[pl.ds(cid * NROW + r0 + j * RCH, RCH)])
        if with_cnt:
            pltpu.sync_copy(cnt.at[pl.ds(r0, RPT)], cbuf)
            pltpu.sync_copy(cbuf, out_cnt.at[pl.ds(cid * NROW + r0, RPT)])

    return pl.kernel(
        body,
        out_type=tuple(out_type) if with_cnt else out_type[0],
        mesh=mesh,
        scratch_types=tuple(scratch),
    )


_agg_with_cnt = _make_agg(True)
_agg = _make_agg(False)


def _dense1_body(a0, a1, c0, c1, x, wl, wr, b, o, oinv):
    c = c0[...] + c1[...]
    ic = 1.0 / jnp.maximum(c, 1.0)
    m = (a0[...] + a1[...]) * ic
    z = (
        jnp.dot(m, wl[...], preferred_element_type=jnp.float32)
        + jnp.dot(x[...], wr[...], preferred_element_type=jnp.float32)
        + b[...]
    )
    o[...] = jnp.maximum(z, 0.0)
    oinv[...] = ic


def _dense_body(act, a0, a1, ic, x, wl, wr, b, o):
    m = (a0[...] + a1[...]) * ic[...]
    z = (
        jnp.dot(m, wl[...], preferred_element_type=jnp.float32)
        + jnp.dot(x[...], wr[...], preferred_element_type=jnp.float32)
        + b[...]
    )
    o[...] = jnp.maximum(z, 0.0) if act else z


_row_spec = pl.BlockSpec((BLK, D), lambda i: (i, 0))
_row1_spec = pl.BlockSpec((BLK, 1), lambda i: (i, 0))
_p0_spec = pl.BlockSpec((BLK, D), lambda i: (i, 0))
_p1_spec = pl.BlockSpec((BLK, D), lambda i: (i + NROW // BLK, 0))
_c0_spec = pl.BlockSpec((BLK, 1), lambda i: (i, 0))
_c1_spec = pl.BlockSpec((BLK, 1), lambda i: (i + NROW // BLK, 0))
_w_spec = pl.BlockSpec((D, D), lambda i: (0, 0))
_b_spec = pl.BlockSpec((1, D), lambda i: (0, 0))

_dense1 = pl.pallas_call(
    _dense1_body,
    grid=(NROW // BLK,),
    in_specs=[_p0_spec, _p1_spec, _c0_spec, _c1_spec, _row_spec, _w_spec, _w_spec, _b_spec],
    out_specs=(_row_spec, _row1_spec),
    out_shape=(
        jax.ShapeDtypeStruct((NROW, D), jnp.float32),
        jax.ShapeDtypeStruct((NROW, 1), jnp.float32),
    ),
)

_dense_relu = pl.pallas_call(
    functools.partial(_dense_body, True),
    grid=(NROW // BLK,),
    in_specs=[_p0_spec, _p1_spec, _row1_spec, _row_spec, _w_spec, _w_spec, _b_spec],
    out_specs=_row_spec,
    out_shape=jax.ShapeDtypeStruct((NROW, D), jnp.float32),
)

_dense_last = pl.pallas_call(
    functools.partial(_dense_body, False),
    grid=(NROW // BLK,),
    in_specs=[_p0_spec, _p1_spec, _row1_spec, _row_spec, _w_spec, _w_spec, _b_spec],
    out_specs=_row_spec,
    out_shape=jax.ShapeDtypeStruct((NROW, D), jnp.float32),
)


def kernel(x, edge_index, Wl1, bl1, Wr1, Wl2, bl2, Wr2, Wl3, bl3, Wr3, g1, b1, g2, b2):
    f32 = jnp.float32
    src = edge_index[0]
    dst = edge_index[1]
    xp = jnp.zeros((NROW, D), f32).at[:N].set(x)

    inv = 1.0 / jnp.sqrt(jnp.asarray(1.0 + 1e-5, f32))
    s1 = g1 * inv
    s2 = g2 * inv
    Wl1t = (Wl1 * s1[:, None]).T
    Wr1t = (Wr1 * s1[:, None]).T
    b1f = (bl1 * s1 + b1).reshape(1, D)
    Wl2t = (Wl2 * s2[:, None]).T
    Wr2t = (Wr2 * s2[:, None]).T
    b2f = (bl2 * s2 + b2).reshape(1, D)
    Wl3t = Wl3.T
    Wr3t = Wr3.T
    b3f = bl3.reshape(1, D)

    agg1, cnt = _agg_with_cnt(xp, src, dst)
    cnt2 = cnt.reshape(NC * NROW, 1)
    h1, invc = _dense1(agg1, agg1, cnt2, cnt2, xp, Wl1t, Wr1t, b1f)
    agg2 = _agg(h1, src, dst)
    h2 = _dense_relu(agg2, agg2, invc, h1, Wl2t, Wr2t, b2f)
    agg3 = _agg(h2, src, dst)
    h3 = _dense_last(agg3, agg3, invc, h2, Wl3t, Wr3t, b3f)
    return h3[:N]


# speedup vs baseline: 14.2440x; 2.6873x over previous
import functools

import jax
import jax.numpy as jnp
from jax import lax
from jax.experimental import pallas as pl
from jax.experimental.pallas import tpu as pltpu
from jax.experimental.pallas import tpu_sc as plsc

N = 10000
E = 320000
D = 128
NC, NS = 2, 16
NW = NC * NS
NROW = 10240
RPT = NROW // NS
EPW = E // NW
CH = 80
NCHUNK = EPW // CH
BLK = 1024


def _zero_2d(buf, rows):
    z16 = jnp.zeros((16,), jnp.float32)

    def body(r, c):
        for j in range(D // 16):
            buf[r, pl.ds(j * 16, 16)] = z16
        return c

    lax.fori_loop(0, rows, body, 0)


def _fill_1d(buf, n, val):
    v16 = jnp.full((16,), val, jnp.float32)

    def body(i, c):
        buf[pl.ds(i * 16, 16)] = v16
        return c

    lax.fori_loop(0, n // 16, body, 0)


NBUF = 4
DEPTH = NBUF - 2
TAIL = (NCHUNK - DEPTH - 1) // NBUF * NBUF


def _make_agg(with_cnt):
    mesh = plsc.VectorSubcoreMesh(
        core_axis_name="c", subcore_axis_name="s", num_cores=NC, num_subcores=NS
    )
    out_type = [jax.ShapeDtypeStruct((NC * NROW, D), jnp.float32)]
    scratch = [
        pltpu.VMEM_SHARED((NROW, D), jnp.float32),
    ]
    scratch += [pltpu.VMEM((CH,), jnp.int32) for _ in range(NBUF)]
    scratch += [pltpu.VMEM((CH,), jnp.int32) for _ in range(NBUF)]
    scratch += [pltpu.VMEM((CH, D), jnp.float32) for _ in range(NBUF)]
    scratch += [pltpu.SemaphoreType.DMA for _ in range(4 * NBUF)]
    if with_cnt:
        out_type.append(jax.ShapeDtypeStruct((NC * NROW,), jnp.float32))
        scratch += [
            pltpu.VMEM_SHARED((NROW,), jnp.float32),
            pltpu.VMEM((CH,), jnp.float32),
            pltpu.VMEM((RPT,), jnp.float32),
        ]

    def body(h, src, dst, *refs):
        if with_cnt:
            out_agg, out_cnt = refs[0], refs[1]
            rest = refs[2:]
        else:
            out_agg = refs[0]
            rest = refs[1:]
        acc = rest[0]
        sidx = rest[1:1 + NBUF]
        didx = rest[1 + NBUF:1 + 2 * NBUF]
        rows = rest[1 + 2 * NBUF:1 + 3 * NBUF]
        gsem = rest[1 + 3 * NBUF:1 + 4 * NBUF]
        ssem = rest[1 + 4 * NBUF:1 + 5 * NBUF]
        dsem = rest[1 + 5 * NBUF:1 + 6 * NBUF]
        xsem = rest[1 + 6 * NBUF:1 + 7 * NBUF]
        if with_cnt:
            cnt, ones, cbuf = rest[1 + 7 * NBUF:4 + 7 * NBUF]
        cid = lax.axis_index("c")
        sid = lax.axis_index("s")
        r0 = sid * RPT
        base = (cid * NS + sid) * EPW

        _zero_2d(rows[0], CH)
        for j in range(RPT // CH):
            pltpu.sync_copy(rows[0], acc.at[pl.ds(r0 + j * CH, CH)])
        if with_cnt:
            _fill_1d(cbuf, RPT, 0.0)
            pltpu.sync_copy(cbuf, cnt.at[pl.ds(r0, RPT)])
            _fill_1d(ones, CH, 1.0)
        plsc.subcore_barrier()

        def load_idx(i, b):
            off = base + i * CH
            pltpu.async_copy(dst.at[pl.ds(off, CH)], didx[b], dsem[b])
            pltpu.async_copy(src.at[pl.ds(off, CH)], sidx[b], xsem[b])

        def wait_didx(b):
            pltpu.make_async_copy(dst.at[pl.ds(0, CH)], didx[b], dsem[b]).wait()

        def wait_sidx(b):
            pltpu.make_async_copy(src.at[pl.ds(0, CH)], sidx[b], xsem[b]).wait()

        def start_gather(i, b):
            pltpu.async_copy(h.at[sidx[b]], rows[b], gsem[b])

        def wait_gather(b):
            pltpu.make_async_copy(h.at[sidx[b]], rows[b], gsem[b]).wait()

        def start_scatter(i, b):
            pltpu.async_copy(rows[b], acc.at[didx[b]], ssem[b], add=True)
            if with_cnt:
                pltpu.async_copy(ones, cnt.at[didx[b]], ssem[b], add=True)

        def wait_scatter(b):
            pltpu.make_async_copy(rows[b], acc.at[didx[b]], ssem[b]).wait()
            if with_cnt:
                pltpu.make_async_copy(ones, cnt.at[didx[b]], ssem[b]).wait()

        for k in range(DEPTH):
            load_idx(k, k)
        for k in range(DEPTH - 1):
            wait_sidx(k)
            start_gather(k, k)

        def emit(j, b, wait_s, load_g, start_g):
            if wait_s:
                wait_scatter((b + DEPTH) % NBUF)
            if load_g:
                load_idx(j + DEPTH, (b + DEPTH) % NBUF)
            if start_g:
                gb = (b + DEPTH - 1) % NBUF
                wait_sidx(gb)
                start_gather(j + DEPTH - 1, gb)
            wait_gather(b)
            wait_didx(b)
            start_scatter(j, b)

        for j in range(NBUF):
            emit(j, j, j >= NBUF - DEPTH, True, True)

        def outer(g, c):
            j0 = g * NBUF
            for b in range(NBUF):
                emit(j0 + b, b, True, True, True)
            return c

        lax.fori_loop(1, TAIL // NBUF, outer, 0)

        for j in range(TAIL, NCHUNK):
            emit(j, j % NBUF, True, j + DEPTH < NCHUNK, j + DEPTH - 1 < NCHUNK)
        for k in range(NCHUNK - (NBUF - DEPTH), NCHUNK):
            wait_scatter(k % NBUF)
        plsc.subcore_barrier()

        for j in range(RPT // CH):
            pltpu.sync_copy(acc.at[pl.ds(r0 + j * CH, CH)], rows[j % 2])
            pltpu.sync_copy(rows[j % 2], out_agg.at[pl.ds(cid * NROW + r0 + j * CH, CH)])
        if with_cnt:
            pltpu.sync_copy(cnt.at[pl.ds(r0, RPT)], cbuf)
            pltpu.sync_copy(cbuf, out_cnt.at[pl.ds(cid * NROW + r0, RPT)])

    return pl.kernel(
        body,
        out_type=tuple(out_type) if with_cnt else out_type[0],
        mesh=mesh,
        scratch_types=tuple(scratch),
    )


_agg_with_cnt = _make_agg(True)
_agg = _make_agg(False)


def _dense1_body(a0, a1, c0, c1, x, wl, wr, b, o, oinv):
    c = c0[...] + c1[...]
    ic = 1.0 / jnp.maximum(c, 1.0)
    m = (a0[...] + a1[...]) * ic
    z = (
        jnp.dot(m, wl[...], preferred_element_type=jnp.float32)
        + jnp.dot(x[...], wr[...], preferred_element_type=jnp.float32)
        + b[...]
    )
    o[...] = jnp.maximum(z, 0.0)
    oinv[...] = ic


def _dense_body(act, a0, a1, ic, x, wl, wr, b, o):
    m = (a0[...] + a1[...]) * ic[...]
    z = (
        jnp.dot(m, wl[...], preferred_element_type=jnp.float32)
        + jnp.dot(x[...], wr[...], preferred_element_type=jnp.float32)
        + b[...]
    )
    o[...] = jnp.maximum(z, 0.0) if act else z


_row_spec = pl.BlockSpec((BLK, D), lambda i: (i, 0))
_row1_spec = pl.BlockSpec((BLK, 1), lambda i: (i, 0))
_p0_spec = pl.BlockSpec((BLK, D), lambda i: (i, 0))
_p1_spec = pl.BlockSpec((BLK, D), lambda i: (i + NROW // BLK, 0))
_c0_spec = pl.BlockSpec((BLK, 1), lambda i: (i, 0))
_c1_spec = pl.BlockSpec((BLK, 1), lambda i: (i + NROW // BLK, 0))
_w_spec = pl.BlockSpec((D, D), lambda i: (0, 0))
_b_spec = pl.BlockSpec((1, D), lambda i: (0, 0))

_dense1 = pl.pallas_call(
    _dense1_body,
    grid=(NROW // BLK,),
    in_specs=[_p0_spec, _p1_spec, _c0_spec, _c1_spec, _row_spec, _w_spec, _w_spec, _b_spec],
    out_specs=(_row_spec, _row1_spec),
    out_shape=(
        jax.ShapeDtypeStruct((NROW, D), jnp.float32),
        jax.ShapeDtypeStruct((NROW, 1), jnp.float32),
    ),
)

_dense_relu = pl.pallas_call(
    functools.partial(_dense_body, True),
    grid=(NROW // BLK,),
    in_specs=[_p0_spec, _p1_spec, _row1_spec, _row_spec, _w_spec, _w_spec, _b_spec],
    out_specs=_row_spec,
    out_shape=jax.ShapeDtypeStruct((NROW, D), jnp.float32),
)

_dense_last = pl.pallas_call(
    functools.partial(_dense_body, False),
    grid=(NROW // BLK,),
    in_specs=[_p0_spec, _p1_spec, _row1_spec, _row_spec, _w_spec, _w_spec, _b_spec],
    out_specs=_row_spec,
    out_shape=jax.ShapeDtypeStruct((NROW, D), jnp.float32),
)


def kernel(x, edge_index, Wl1, bl1, Wr1, Wl2, bl2, Wr2, Wl3, bl3, Wr3, g1, b1, g2, b2):
    f32 = jnp.float32
    src = edge_index[0]
    dst = edge_index[1]
    xp = jnp.zeros((NROW, D), f32).at[:N].set(x)

    inv = 1.0 / jnp.sqrt(jnp.asarray(1.0 + 1e-5, f32))
    s1 = g1 * inv
    s2 = g2 * inv
    Wl1t = (Wl1 * s1[:, None]).T
    Wr1t = (Wr1 * s1[:, None]).T
    b1f = (bl1 * s1 + b1).reshape(1, D)
    Wl2t = (Wl2 * s2[:, None]).T
    Wr2t = (Wr2 * s2[:, None]).T
    b2f = (bl2 * s2 + b2).reshape(1, D)
    Wl3t = Wl3.T
    Wr3t = Wr3.T
    b3f = bl3.reshape(1, D)

    agg1, cnt = _agg_with_cnt(xp, src, dst)
    cnt2 = cnt.reshape(NC * NROW, 1)
    h1, invc = _dense1(agg1, agg1, cnt2, cnt2, xp, Wl1t, Wr1t, b1f)
    agg2 = _agg(h1, src, dst)
    h2 = _dense_relu(agg2, agg2, invc, h1, Wl2t, Wr2t, b2f)
    agg3 = _agg(h2, src, dst)
    h3 = _dense_last(agg3, agg3, invc, h2, Wl3t, Wr3t, b3f)
    return h3[:N]


# speedup vs baseline: 14.7011x; 1.0321x over previous
import functools

import jax
import jax.numpy as jnp
from jax import lax
from jax.experimental import pallas as pl
from jax.experimental.pallas import tpu as pltpu
from jax.experimental.pallas import tpu_sc as plsc

N = 10000
E = 320000
D = 128
NC, NS = 2, 16
NW = NC * NS
NROW = 10240
RPT = NROW // NS
EPW = E // NW
CH = 80
NCHUNK = EPW // CH
BLK = 1024


def _zero_2d(buf, rows):
    z16 = jnp.zeros((16,), jnp.float32)

    def body(r, c):
        for j in range(D // 16):
            buf[r, pl.ds(j * 16, 16)] = z16
        return c

    lax.fori_loop(0, rows, body, 0)


def _fill_1d(buf, n, val):
    v16 = jnp.full((16,), val, jnp.float32)

    def body(i, c):
        buf[pl.ds(i * 16, 16)] = v16
        return c

    lax.fori_loop(0, n // 16, body, 0)


NBUF = 4
DEPTH = NBUF - 1
TAIL = (NCHUNK - DEPTH - 1) // NBUF * NBUF


def _make_agg(with_cnt):
    mesh = plsc.VectorSubcoreMesh(
        core_axis_name="c", subcore_axis_name="s", num_cores=NC, num_subcores=NS
    )
    out_type = [jax.ShapeDtypeStruct((NC * NROW, D), jnp.float32)]
    scratch = [
        pltpu.VMEM_SHARED((NROW, D), jnp.float32),
    ]
    scratch += [pltpu.VMEM((CH,), jnp.int32) for _ in range(NBUF)]
    scratch += [pltpu.VMEM((CH,), jnp.int32) for _ in range(NBUF)]
    scratch += [pltpu.VMEM((CH, D), jnp.float32) for _ in range(NBUF)]
    scratch += [pltpu.SemaphoreType.DMA for _ in range(4 * NBUF)]
    if with_cnt:
        out_type.append(jax.ShapeDtypeStruct((NC * NROW,), jnp.float32))
        scratch += [
            pltpu.VMEM_SHARED((NROW,), jnp.float32),
            pltpu.VMEM((CH,), jnp.float32),
            pltpu.VMEM((RPT,), jnp.float32),
        ]

    def body(h, src, dst, *refs):
        if with_cnt:
            out_agg, out_cnt = refs[0], refs[1]
            rest = refs[2:]
        else:
            out_agg = refs[0]
            rest = refs[1:]
        acc = rest[0]
        sidx = rest[1:1 + NBUF]
        didx = rest[1 + NBUF:1 + 2 * NBUF]
        rows = rest[1 + 2 * NBUF:1 + 3 * NBUF]
        gsem = rest[1 + 3 * NBUF:1 + 4 * NBUF]
        ssem = rest[1 + 4 * NBUF:1 + 5 * NBUF]
        dsem = rest[1 + 5 * NBUF:1 + 6 * NBUF]
        xsem = rest[1 + 6 * NBUF:1 + 7 * NBUF]
        if with_cnt:
            cnt, ones, cbuf = rest[1 + 7 * NBUF:4 + 7 * NBUF]
        cid = lax.axis_index("c")
        sid = lax.axis_index("s")
        r0 = sid * RPT
        base = (cid * NS + sid) * EPW

        _zero_2d(rows[0], CH)
        for j in range(RPT // CH):
            pltpu.sync_copy(rows[0], acc.at[pl.ds(r0 + j * CH, CH)])
        if with_cnt:
            _fill_1d(cbuf, RPT, 0.0)
            pltpu.sync_copy(cbuf, cnt.at[pl.ds(r0, RPT)])
            _fill_1d(ones, CH, 1.0)
        plsc.subcore_barrier()

        def load_idx(i, b):
            off = base + i * CH
            pltpu.async_copy(dst.at[pl.ds(off, CH)], didx[b], dsem[b])
            pltpu.async_copy(src.at[pl.ds(off, CH)], sidx[b], xsem[b])

        def wait_didx(b):
            pltpu.make_async_copy(dst.at[pl.ds(0, CH)], didx[b], dsem[b]).wait()

        def wait_sidx(b):
            pltpu.make_async_copy(src.at[pl.ds(0, CH)], sidx[b], xsem[b]).wait()

        def start_gather(i, b):
            pltpu.async_copy(h.at[sidx[b]], rows[b], gsem[b])

        def wait_gather(b):
            pltpu.make_async_copy(h.at[sidx[b]], rows[b], gsem[b]).wait()

        def start_scatter(i, b):
            pltpu.async_copy(rows[b], acc.at[didx[b]], ssem[b], add=True)
            if with_cnt:
                pltpu.async_copy(ones, cnt.at[didx[b]], ssem[b], add=True)

        def wait_scatter(b):
            pltpu.make_async_copy(rows[b], acc.at[didx[b]], ssem[b]).wait()
            if with_cnt:
                pltpu.make_async_copy(ones, cnt.at[didx[b]], ssem[b]).wait()

        for k in range(DEPTH):
            load_idx(k, k)
        for k in range(DEPTH - 1):
            wait_sidx(k)
            start_gather(k, k)

        def emit(j, b, wait_s, load_g, start_g):
            if wait_s:
                wait_scatter((b + DEPTH) % NBUF)
            if load_g:
                load_idx(j + DEPTH, (b + DEPTH) % NBUF)
            if start_g:
                gb = (b + DEPTH - 1) % NBUF
                wait_sidx(gb)
                start_gather(j + DEPTH - 1, gb)
            wait_gather(b)
            wait_didx(b)
            start_scatter(j, b)

        for j in range(NBUF):
            emit(j, j, j >= NBUF - DEPTH, True, True)

        def outer(g, c):
            j0 = g * NBUF
            for b in range(NBUF):
                emit(j0 + b, b, True, True, True)
            return c

        lax.fori_loop(1, TAIL // NBUF, outer, 0)

        for j in range(TAIL, NCHUNK):
            emit(j, j % NBUF, True, j + DEPTH < NCHUNK, j + DEPTH - 1 < NCHUNK)
        for k in range(NCHUNK - (NBUF - DEPTH), NCHUNK):
            wait_scatter(k % NBUF)
        plsc.subcore_barrier()

        for j in range(RPT // CH):
            pltpu.sync_copy(acc.at[pl.ds(r0 + j * CH, CH)], rows[j % 2])
            pltpu.sync_copy(rows[j % 2], out_agg.at[pl.ds(cid * NROW + r0 + j * CH, CH)])
        if with_cnt:
            pltpu.sync_copy(cnt.at[pl.ds(r0, RPT)], cbuf)
            pltpu.sync_copy(cbuf, out_cnt.at[pl.ds(cid * NROW + r0, RPT)])

    return pl.kernel(
        body,
        out_type=tuple(out_type) if with_cnt else out_type[0],
        mesh=mesh,
        scratch_types=tuple(scratch),
    )


_agg_with_cnt = _make_agg(True)
_agg = _make_agg(False)


def _dense1_body(a0, a1, c0, c1, x, wl, wr, b, o, oinv):
    c = c0[...] + c1[...]
    ic = 1.0 / jnp.maximum(c, 1.0)
    m = (a0[...] + a1[...]) * ic
    z = (
        jnp.dot(m, wl[...], preferred_element_type=jnp.float32)
        + jnp.dot(x[...], wr[...], preferred_element_type=jnp.float32)
        + b[...]
    )
    o[...] = jnp.maximum(z, 0.0)
    oinv[...] = ic


def _dense_body(act, a0, a1, ic, x, wl, wr, b, o):
    m = (a0[...] + a1[...]) * ic[...]
    z = (
        jnp.dot(m, wl[...], preferred_element_type=jnp.float32)
        + jnp.dot(x[...], wr[...], preferred_element_type=jnp.float32)
        + b[...]
    )
    o[...] = jnp.maximum(z, 0.0) if act else z


_row_spec = pl.BlockSpec((BLK, D), lambda i: (i, 0))
_row1_spec = pl.BlockSpec((BLK, 1), lambda i: (i, 0))
_p0_spec = pl.BlockSpec((BLK, D), lambda i: (i, 0))
_p1_spec = pl.BlockSpec((BLK, D), lambda i: (i + NROW // BLK, 0))
_c0_spec = pl.BlockSpec((BLK, 1), lambda i: (i, 0))
_c1_spec = pl.BlockSpec((BLK, 1), lambda i: (i + NROW // BLK, 0))
_w_spec = pl.BlockSpec((D, D), lambda i: (0, 0))
_b_spec = pl.BlockSpec((1, D), lambda i: (0, 0))

_dense1 = pl.pallas_call(
    _dense1_body,
    grid=(NROW // BLK,),
    in_specs=[_p0_spec, _p1_spec, _c0_spec, _c1_spec, _row_spec, _w_spec, _w_spec, _b_spec],
    out_specs=(_row_spec, _row1_spec),
    out_shape=(
        jax.ShapeDtypeStruct((NROW, D), jnp.float32),
        jax.ShapeDtypeStruct((NROW, 1), jnp.float32),
    ),
)

_dense_relu = pl.pallas_call(
    functools.partial(_dense_body, True),
    grid=(NROW // BLK,),
    in_specs=[_p0_spec, _p1_spec, _row1_spec, _row_spec, _w_spec, _w_spec, _b_spec],
    out_specs=_row_spec,
    out_shape=jax.ShapeDtypeStruct((NROW, D), jnp.float32),
)

_dense_last = pl.pallas_call(
    functools.partial(_dense_body, False),
    grid=(NROW // BLK,),
    in_specs=[_p0_spec, _p1_spec, _row1_spec, _row_spec, _w_spec, _w_spec, _b_spec],
    out_specs=_row_spec,
    out_shape=jax.ShapeDtypeStruct((NROW, D), jnp.float32),
)


def kernel(x, edge_index, Wl1, bl1, Wr1, Wl2, bl2, Wr2, Wl3, bl3, Wr3, g1, b1, g2, b2):
    f32 = jnp.float32
    src = edge_index[0]
    dst = edge_index[1]
    xp = jnp.zeros((NROW, D), f32).at[:N].set(x)

    inv = 1.0 / jnp.sqrt(jnp.asarray(1.0 + 1e-5, f32))
    s1 = g1 * inv
    s2 = g2 * inv
    Wl1t = (Wl1 * s1[:, None]).T
    Wr1t = (Wr1 * s1[:, None]).T
    b1f = (bl1 * s1 + b1).reshape(1, D)
    Wl2t = (Wl2 * s2[:, None]).T
    Wr2t = (Wr2 * s2[:, None]).T
    b2f = (bl2 * s2 + b2).reshape(1, D)
    Wl3t = Wl3.T
    Wr3t = Wr3.T
    b3f = bl3.reshape(1, D)

    agg1, cnt = _agg_with_cnt(xp, src, dst)
    cnt2 = cnt.reshape(NC * NROW, 1)
    h1, invc = _dense1(agg1, agg1, cnt2, cnt2, xp, Wl1t, Wr1t, b1f)
    agg2 = _agg(h1, src, dst)
    h2 = _dense_relu(agg2, agg2, invc, h1, Wl2t, Wr2t, b2f)
    agg3 = _agg(h2, src, dst)
    h3 = _dense_last(agg3, agg3, invc, h2, Wl3t, Wr3t, b3f)
    return h3[:N]


# speedup vs baseline: 15.2021x; 1.0341x over previous
import functools

import jax
import jax.numpy as jnp
from jax import lax
from jax.experimental import pallas as pl
from jax.experimental.pallas import tpu as pltpu
from jax.experimental.pallas import tpu_sc as plsc

N = 10000
E = 320000
D = 128
NC, NS = 2, 16
NW = NC * NS
NROW = 10240
RPT = NROW // NS
EPW = E // NW
CH = 80
NCHUNK = EPW // CH
BLK = 1024


def _zero_2d(buf, rows):
    z16 = jnp.zeros((16,), jnp.float32)

    def body(r, c):
        for j in range(D // 16):
            buf[r, pl.ds(j * 16, 16)] = z16
        return c

    lax.fori_loop(0, rows, body, 0)


def _fill_1d(buf, n, val):
    v16 = jnp.full((16,), val, jnp.float32)

    def body(i, c):
        buf[pl.ds(i * 16, 16)] = v16
        return c

    lax.fori_loop(0, n // 16, body, 0)


NBUF = 4
DEPTH = NBUF - 1
TAIL = (NCHUNK - DEPTH - 1) // NBUF * NBUF


def _make_agg(with_cnt):
    mesh = plsc.VectorSubcoreMesh(
        core_axis_name="c", subcore_axis_name="s", num_cores=NC, num_subcores=NS
    )
    out_type = [jax.ShapeDtypeStruct((NC * NROW, D), jnp.float32)]
    scratch = [
        pltpu.VMEM_SHARED((NROW, D), jnp.float32),
    ]
    scratch += [pltpu.VMEM((CH,), jnp.int32) for _ in range(NBUF)]
    scratch += [pltpu.VMEM((CH,), jnp.int32) for _ in range(NBUF)]
    scratch += [pltpu.VMEM((CH, D), jnp.float32) for _ in range(NBUF)]
    scratch += [pltpu.SemaphoreType.DMA for _ in range(4 * NBUF)]
    if with_cnt:
        out_type.append(jax.ShapeDtypeStruct((NC * NROW,), jnp.float32))
        scratch += [
            pltpu.VMEM_SHARED((NROW,), jnp.float32),
            pltpu.VMEM((CH,), jnp.float32),
            pltpu.VMEM((RPT,), jnp.float32),
        ]

    def body(h, src, dst, *refs):
        if with_cnt:
            out_agg, out_cnt = refs[0], refs[1]
            rest = refs[2:]
        else:
            out_agg = refs[0]
            rest = refs[1:]
        acc = rest[0]
        sidx = rest[1:1 + NBUF]
        didx = rest[1 + NBUF:1 + 2 * NBUF]
        rows = rest[1 + 2 * NBUF:1 + 3 * NBUF]
        gsem = rest[1 + 3 * NBUF:1 + 4 * NBUF]
        ssem = rest[1 + 4 * NBUF:1 + 5 * NBUF]
        dsem = rest[1 + 5 * NBUF:1 + 6 * NBUF]
        xsem = rest[1 + 6 * NBUF:1 + 7 * NBUF]
        if with_cnt:
            cnt, ones, cbuf = rest[1 + 7 * NBUF:4 + 7 * NBUF]
        cid = lax.axis_index("c")
        sid = lax.axis_index("s")
        r0 = sid * RPT
        base = (cid * NS + sid) * EPW

        def load_idx(i, b):
            off = base + i * CH
            pltpu.async_copy(dst.at[pl.ds(off, CH)], didx[b], dsem[b])
            pltpu.async_copy(src.at[pl.ds(off, CH)], sidx[b], xsem[b])

        def wait_didx(b):
            pltpu.make_async_copy(dst.at[pl.ds(0, CH)], didx[b], dsem[b]).wait()

        def wait_sidx(b):
            pltpu.make_async_copy(src.at[pl.ds(0, CH)], sidx[b], xsem[b]).wait()

        def start_gather(i, b):
            pltpu.async_copy(h.at[sidx[b]], rows[b], gsem[b])

        def wait_gather(b):
            pltpu.make_async_copy(h.at[sidx[b]], rows[b], gsem[b]).wait()

        def start_scatter(i, b):
            pltpu.async_copy(rows[b], acc.at[didx[b]], ssem[b], add=True)
            if with_cnt:
                pltpu.async_copy(ones, cnt.at[didx[b]], ssem[b], add=True)

        def wait_scatter(b):
            pltpu.make_async_copy(rows[b], acc.at[didx[b]], ssem[b]).wait()
            if with_cnt:
                pltpu.make_async_copy(ones, cnt.at[didx[b]], ssem[b]).wait()

        for k in range(DEPTH):
            load_idx(k, k)
        for k in range(DEPTH - 1):
            wait_sidx(k)
            start_gather(k, k)

        _zero_2d(rows[NBUF - 1], CH)
        for j in range(RPT // CH):
            pltpu.sync_copy(rows[NBUF - 1], acc.at[pl.ds(r0 + j * CH, CH)])
        if with_cnt:
            _fill_1d(cbuf, RPT, 0.0)
            pltpu.sync_copy(cbuf, cnt.at[pl.ds(r0, RPT)])
            _fill_1d(ones, CH, 1.0)
        plsc.subcore_barrier()

        def emit(j, b, wait_s, load_g, start_g):
            if wait_s:
                wait_scatter((b + DEPTH) % NBUF)
            if load_g:
                load_idx(j + DEPTH, (b + DEPTH) % NBUF)
            if start_g:
                gb = (b + DEPTH - 1) % NBUF
                wait_sidx(gb)
                start_gather(j + DEPTH - 1, gb)
            wait_gather(b)
            wait_didx(b)
            start_scatter(j, b)

        for j in range(NBUF):
            emit(j, j, j >= NBUF - DEPTH, True, True)

        def outer(g, c):
            j0 = g * NBUF
            for b in range(NBUF):
                emit(j0 + b, b, True, True, True)
            return c

        lax.fori_loop(1, TAIL // NBUF, outer, 0)

        for j in range(TAIL, NCHUNK):
            emit(j, j % NBUF, True, j + DEPTH < NCHUNK, j + DEPTH - 1 < NCHUNK)
        for k in range(NCHUNK - (NBUF - DEPTH), NCHUNK):
            wait_scatter(k % NBUF)
        plsc.subcore_barrier()

        def _ob(j):
            return out_agg.at[pl.ds(cid * NROW + r0 + j * CH, CH)]

        for j in range(RPT // CH):
            b = j % 2
            if j >= 2:
                pltpu.make_async_copy(rows[b], _ob(j - 2), ssem[b]).wait()
            pltpu.sync_copy(acc.at[pl.ds(r0 + j * CH, CH)], rows[b])
            pltpu.async_copy(rows[b], _ob(j), ssem[b])
        if with_cnt:
            pltpu.sync_copy(cnt.at[pl.ds(r0, RPT)], cbuf)
            pltpu.sync_copy(cbuf, out_cnt.at[pl.ds(cid * NROW + r0, RPT)])
        for j in range(RPT // CH - 2, RPT // CH):
            pltpu.make_async_copy(rows[j % 2], _ob(j), ssem[j % 2]).wait()

    return pl.kernel(
        body,
        out_type=tuple(out_type) if with_cnt else out_type[0],
        mesh=mesh,
        scratch_types=tuple(scratch),
    )


_agg_with_cnt = _make_agg(True)
_agg = _make_agg(False)


def _dense1_body(a0, a1, c0, c1, x, wl, wr, b, o, oinv):
    c = c0[...] + c1[...]
    ic = 1.0 / jnp.maximum(c, 1.0)
    m = (a0[...] + a1[...]) * ic
    z = (
        jnp.dot(m, wl[...], preferred_element_type=jnp.float32)
        + jnp.dot(x[...], wr[...], preferred_element_type=jnp.float32)
        + b[...]
    )
    o[...] = jnp.maximum(z, 0.0)
    oinv[...] = ic


def _dense_body(act, a0, a1, ic, x, wl, wr, b, o):
    m = (a0[...] + a1[...]) * ic[...]
    z = (
        jnp.dot(m, wl[...], preferred_element_type=jnp.float32)
        + jnp.dot(x[...], wr[...], preferred_element_type=jnp.float32)
        + b[...]
    )
    o[...] = jnp.maximum(z, 0.0) if act else z


_row_spec = pl.BlockSpec((BLK, D), lambda i: (i, 0))
_row1_spec = pl.BlockSpec((BLK, 1), lambda i: (i, 0))
_p0_spec = pl.BlockSpec((BLK, D), lambda i: (i, 0))
_p1_spec = pl.BlockSpec((BLK, D), lambda i: (i + NROW // BLK, 0))
_c0_spec = pl.BlockSpec((BLK, 1), lambda i: (i, 0))
_c1_spec = pl.BlockSpec((BLK, 1), lambda i: (i + NROW // BLK, 0))
_w_spec = pl.BlockSpec((D, D), lambda i: (0, 0))
_b_spec = pl.BlockSpec((1, D), lambda i: (0, 0))

_dense1 = pl.pallas_call(
    _dense1_body,
    grid=(NROW // BLK,),
    in_specs=[_p0_spec, _p1_spec, _c0_spec, _c1_spec, _row_spec, _w_spec, _w_spec, _b_spec],
    out_specs=(_row_spec, _row1_spec),
    out_shape=(
        jax.ShapeDtypeStruct((NROW, D), jnp.float32),
        jax.ShapeDtypeStruct((NROW, 1), jnp.float32),
    ),
)

_dense_relu = pl.pallas_call(
    functools.partial(_dense_body, True),
    grid=(NROW // BLK,),
    in_specs=[_p0_spec, _p1_spec, _row1_spec, _row_spec, _w_spec, _w_spec, _b_spec],
    out_specs=_row_spec,
    out_shape=jax.ShapeDtypeStruct((NROW, D), jnp.float32),
)

_dense_last = pl.pallas_call(
    functools.partial(_dense_body, False),
    grid=(NROW // BLK,),
    in_specs=[_p0_spec, _p1_spec, _row1_spec, _row_spec, _w_spec, _w_spec, _b_spec],
    out_specs=_row_spec,
    out_shape=jax.ShapeDtypeStruct((NROW, D), jnp.float32),
)


def kernel(x, edge_index, Wl1, bl1, Wr1, Wl2, bl2, Wr2, Wl3, bl3, Wr3, g1, b1, g2, b2):
    f32 = jnp.float32
    src = edge_index[0]
    dst = edge_index[1]
    xp = jnp.zeros((NROW, D), f32).at[:N].set(x)

    inv = 1.0 / jnp.sqrt(jnp.asarray(1.0 + 1e-5, f32))
    s1 = g1 * inv
    s2 = g2 * inv
    Wl1t = (Wl1 * s1[:, None]).T
    Wr1t = (Wr1 * s1[:, None]).T
    b1f = (bl1 * s1 + b1).reshape(1, D)
    Wl2t = (Wl2 * s2[:, None]).T
    Wr2t = (Wr2 * s2[:, None]).T
    b2f = (bl2 * s2 + b2).reshape(1, D)
    Wl3t = Wl3.T
    Wr3t = Wr3.T
    b3f = bl3.reshape(1, D)

    agg1, cnt = _agg_with_cnt(xp, src, dst)
    cnt2 = cnt.reshape(NC * NROW, 1)
    h1, invc = _dense1(agg1, agg1, cnt2, cnt2, xp, Wl1t, Wr1t, b1f)
    agg2 = _agg(h1, src, dst)
    h2 = _dense_relu(agg2, agg2, invc, h1, Wl2t, Wr2t, b2f)
    agg3 = _agg(h2, src, dst)
    h3 = _dense_last(agg3, agg3, invc, h2, Wl3t, Wr3t, b3f)
    return h3[:N]


# speedup vs baseline: 15.2346x; 1.0021x over previous
import functools

import jax
import jax.numpy as jnp
from jax import lax
from jax.experimental import pallas as pl
from jax.experimental.pallas import tpu as pltpu
from jax.experimental.pallas import tpu_sc as plsc

N = 10000
E = 320000
D = 128
NC, NS = 2, 16
NW = NC * NS
NROW = 10240
RPT = NROW // NS
EPW = E // NW
CH = 80
NCHUNK = EPW // CH
BLK = 1000


def _zero_2d(buf, rows):
    z16 = jnp.zeros((16,), jnp.float32)

    def body(r, c):
        for j in range(D // 16):
            buf[r, pl.ds(j * 16, 16)] = z16
        return c

    lax.fori_loop(0, rows, body, 0)


def _fill_1d(buf, n, val):
    v16 = jnp.full((16,), val, jnp.float32)

    def body(i, c):
        buf[pl.ds(i * 16, 16)] = v16
        return c

    lax.fori_loop(0, n // 16, body, 0)


NBUF = 4
DEPTH = NBUF - 1
TAIL = (NCHUNK - DEPTH - 1) // NBUF * NBUF


def _make_agg(with_cnt):
    mesh = plsc.VectorSubcoreMesh(
        core_axis_name="c", subcore_axis_name="s", num_cores=NC, num_subcores=NS
    )
    out_type = [jax.ShapeDtypeStruct((NC * NROW, D), jnp.float32)]
    scratch = [
        pltpu.VMEM_SHARED((NROW, D), jnp.float32),
    ]
    scratch += [pltpu.VMEM((CH,), jnp.int32) for _ in range(NBUF)]
    scratch += [pltpu.VMEM((CH,), jnp.int32) for _ in range(NBUF)]
    scratch += [pltpu.VMEM((CH, D), jnp.float32) for _ in range(NBUF)]
    scratch += [pltpu.SemaphoreType.DMA for _ in range(4 * NBUF)]
    if with_cnt:
        out_type.append(jax.ShapeDtypeStruct((NC * NROW,), jnp.float32))
        scratch += [
            pltpu.VMEM_SHARED((NROW,), jnp.float32),
            pltpu.VMEM((CH,), jnp.float32),
            pltpu.VMEM((RPT,), jnp.float32),
        ]

    def body(h, src, dst, *refs):
        if with_cnt:
            out_agg, out_cnt = refs[0], refs[1]
            rest = refs[2:]
        else:
            out_agg = refs[0]
            rest = refs[1:]
        acc = rest[0]
        sidx = rest[1:1 + NBUF]
        didx = rest[1 + NBUF:1 + 2 * NBUF]
        rows = rest[1 + 2 * NBUF:1 + 3 * NBUF]
        gsem = rest[1 + 3 * NBUF:1 + 4 * NBUF]
        ssem = rest[1 + 4 * NBUF:1 + 5 * NBUF]
        dsem = rest[1 + 5 * NBUF:1 + 6 * NBUF]
        xsem = rest[1 + 6 * NBUF:1 + 7 * NBUF]
        if with_cnt:
            cnt, ones, cbuf = rest[1 + 7 * NBUF:4 + 7 * NBUF]
        cid = lax.axis_index("c")
        sid = lax.axis_index("s")
        r0 = sid * RPT
        base = (cid * NS + sid) * EPW

        def load_idx(i, b):
            off = base + i * CH
            pltpu.async_copy(dst.at[pl.ds(off, CH)], didx[b], dsem[b])
            pltpu.async_copy(src.at[pl.ds(off, CH)], sidx[b], xsem[b])

        def wait_didx(b):
            pltpu.make_async_copy(dst.at[pl.ds(0, CH)], didx[b], dsem[b]).wait()

        def wait_sidx(b):
            pltpu.make_async_copy(src.at[pl.ds(0, CH)], sidx[b], xsem[b]).wait()

        def start_gather(i, b):
            pltpu.async_copy(h.at[sidx[b]], rows[b], gsem[b])

        def wait_gather(b):
            pltpu.make_async_copy(h.at[sidx[b]], rows[b], gsem[b]).wait()

        def start_scatter(i, b):
            pltpu.async_copy(rows[b], acc.at[didx[b]], ssem[b], add=True)
            if with_cnt:
                pltpu.async_copy(ones, cnt.at[didx[b]], ssem[b], add=True)

        def wait_scatter(b):
            pltpu.make_async_copy(rows[b], acc.at[didx[b]], ssem[b]).wait()
            if with_cnt:
                pltpu.make_async_copy(ones, cnt.at[didx[b]], ssem[b]).wait()

        for k in range(DEPTH):
            load_idx(k, k)
        for k in range(DEPTH - 1):
            wait_sidx(k)
            start_gather(k, k)

        _zero_2d(rows[NBUF - 1], CH)
        for j in range(RPT // CH):
            pltpu.sync_copy(rows[NBUF - 1], acc.at[pl.ds(r0 + j * CH, CH)])
        if with_cnt:
            _fill_1d(cbuf, RPT, 0.0)
            pltpu.sync_copy(cbuf, cnt.at[pl.ds(r0, RPT)])
            _fill_1d(ones, CH, 1.0)
        plsc.subcore_barrier()

        def emit(j, b, wait_s, load_g, start_g):
            if wait_s:
                wait_scatter((b + DEPTH) % NBUF)
            if load_g:
                load_idx(j + DEPTH, (b + DEPTH) % NBUF)
            if start_g:
                gb = (b + DEPTH - 1) % NBUF
                wait_sidx(gb)
                start_gather(j + DEPTH - 1, gb)
            wait_gather(b)
            wait_didx(b)
            start_scatter(j, b)

        for j in range(NBUF):
            emit(j, j, j >= NBUF - DEPTH, True, True)

        def outer(g, c):
            j0 = g * NBUF
            for b in range(NBUF):
                emit(j0 + b, b, True, True, True)
            return c

        lax.fori_loop(1, TAIL // NBUF, outer, 0)

        for j in range(TAIL, NCHUNK):
            emit(j, j % NBUF, True, j + DEPTH < NCHUNK, j + DEPTH - 1 < NCHUNK)
        for k in range(NCHUNK - (NBUF - DEPTH), NCHUNK):
            wait_scatter(k % NBUF)
        plsc.subcore_barrier()

        def _ob(j):
            return out_agg.at[pl.ds(cid * NROW + r0 + j * CH, CH)]

        for j in range(RPT // CH):
            b = j % 2
            if j >= 2:
                pltpu.make_async_copy(rows[b], _ob(j - 2), ssem[b]).wait()
            pltpu.sync_copy(acc.at[pl.ds(r0 + j * CH, CH)], rows[b])
            pltpu.async_copy(rows[b], _ob(j), ssem[b])
        if with_cnt:
            pltpu.sync_copy(cnt.at[pl.ds(r0, RPT)], cbuf)
            pltpu.sync_copy(cbuf, out_cnt.at[pl.ds(cid * NROW + r0, RPT)])
        for j in range(RPT // CH - 2, RPT // CH):
            pltpu.make_async_copy(rows[j % 2], _ob(j), ssem[j % 2]).wait()

    return pl.kernel(
        body,
        out_type=tuple(out_type) if with_cnt else out_type[0],
        mesh=mesh,
        scratch_types=tuple(scratch),
    )


_agg_with_cnt = _make_agg(True)
_agg = _make_agg(False)


def _dense1_body(a0, a1, c0, c1, x, wl, wr, b, o, oinv):
    c = c0[0] + c1[0]
    ic = 1.0 / jnp.maximum(c, 1.0)
    m = (a0[0] + a1[0]) * ic
    z = (
        jnp.dot(m, wl[...], preferred_element_type=jnp.float32)
        + jnp.dot(x[...], wr[...], preferred_element_type=jnp.float32)
        + b[...]
    )
    o[...] = jnp.maximum(z, 0.0)
    oinv[...] = ic


def _dense_body(act, a0, a1, ic, x, wl, wr, b, o):
    m = (a0[0] + a1[0]) * ic[...]
    z = (
        jnp.dot(m, wl[...], preferred_element_type=jnp.float32)
        + jnp.dot(x[...], wr[...], preferred_element_type=jnp.float32)
        + b[...]
    )
    o[...] = jnp.maximum(z, 0.0) if act else z


_row_spec = pl.BlockSpec((BLK, D), lambda i: (i, 0))
_row1_spec = pl.BlockSpec((BLK, 1), lambda i: (i, 0))
_p0_spec = pl.BlockSpec((1, BLK, D), lambda i: (0, i, 0))
_p1_spec = pl.BlockSpec((1, BLK, D), lambda i: (1, i, 0))
_c0_spec = pl.BlockSpec((1, BLK, 1), lambda i: (0, i, 0))
_c1_spec = pl.BlockSpec((1, BLK, 1), lambda i: (1, i, 0))
_w_spec = pl.BlockSpec((D, D), lambda i: (0, 0))
_b_spec = pl.BlockSpec((1, D), lambda i: (0, 0))

_dense1 = pl.pallas_call(
    _dense1_body,
    grid=(N // BLK,),
    in_specs=[_p0_spec, _p1_spec, _c0_spec, _c1_spec, _row_spec, _w_spec, _w_spec, _b_spec],
    out_specs=(_row_spec, _row1_spec),
    out_shape=(
        jax.ShapeDtypeStruct((N, D), jnp.float32),
        jax.ShapeDtypeStruct((N, 1), jnp.float32),
    ),
)

_dense_relu = pl.pallas_call(
    functools.partial(_dense_body, True),
    grid=(N // BLK,),
    in_specs=[_p0_spec, _p1_spec, _row1_spec, _row_spec, _w_spec, _w_spec, _b_spec],
    out_specs=_row_spec,
    out_shape=jax.ShapeDtypeStruct((N, D), jnp.float32),
)

_dense_last = pl.pallas_call(
    functools.partial(_dense_body, False),
    grid=(N // BLK,),
    in_specs=[_p0_spec, _p1_spec, _row1_spec, _row_spec, _w_spec, _w_spec, _b_spec],
    out_specs=_row_spec,
    out_shape=jax.ShapeDtypeStruct((N, D), jnp.float32),
)


def kernel(x, edge_index, Wl1, bl1, Wr1, Wl2, bl2, Wr2, Wl3, bl3, Wr3, g1, b1, g2, b2):
    f32 = jnp.float32
    src = edge_index[0]
    dst = edge_index[1]

    inv = 1.0 / jnp.sqrt(jnp.asarray(1.0 + 1e-5, f32))
    s1 = g1 * inv
    s2 = g2 * inv
    Wl1t = (Wl1 * s1[:, None]).T
    Wr1t = (Wr1 * s1[:, None]).T
    b1f = (bl1 * s1 + b1).reshape(1, D)
    Wl2t = (Wl2 * s2[:, None]).T
    Wr2t = (Wr2 * s2[:, None]).T
    b2f = (bl2 * s2 + b2).reshape(1, D)
    Wl3t = Wl3.T
    Wr3t = Wr3.T
    b3f = bl3.reshape(1, D)

    agg1, cnt = _agg_with_cnt(x, src, dst)
    a1v = agg1.reshape(NC, NROW, D)
    cv = cnt.reshape(NC, NROW, 1)
    h1, invc = _dense1(a1v, a1v, cv, cv, x, Wl1t, Wr1t, b1f)
    a2v = _agg(h1, src, dst).reshape(NC, NROW, D)
    h2 = _dense_relu(a2v, a2v, invc, h1, Wl2t, Wr2t, b2f)
    a3v = _agg(h2, src, dst).reshape(NC, NROW, D)
    return _dense_last(a3v, a3v, invc, h2, Wl3t, Wr3t, b3f)
